# baseline (speedup 1.0000x reference)
import functools

import jax
import jax.numpy as jnp
from jax import lax
from jax.experimental import pallas as pl
from jax.experimental.pallas import tpu as pltpu
from jax.experimental.pallas import tpu_sc as plsc

_NC = 2
_NS = 16
_NW = _NC * _NS

_CW = 128


def _sc_gather_fn(ch):
    mesh = plsc.VectorSubcoreMesh(core_axis_name="c", subcore_axis_name="s")

    @functools.partial(
        pl.kernel,
        mesh=mesh,
        out_type=jax.ShapeDtypeStruct((_NW, ch, _CW), jnp.float32),
        scratch_types=[
            pltpu.VMEM((ch, _CW), jnp.int32),
            pltpu.VMEM((ch, _CW), jnp.float32),
            pltpu.SemaphoreType.DMA,
        ],
    )
    def sc_gather(a_hbm, idx_hbm, out_hbm, idx_v, val_v, sem):
        wid = lax.axis_index("s") * _NC + lax.axis_index("c")
        pltpu.sync_copy(idx_hbm.at[wid], idx_v)

        def row(r, _):
            copies = [
                pltpu.async_copy(
                    a_hbm.at[idx_v[r, pl.ds(k * 16, 16)]],
                    val_v.at[r, pl.ds(k * 16, 16)],
                    sem,
                )
                for k in range(_CW // 16)
            ]
            for c in copies:
                c.wait()
            return 0

        lax.fori_loop(0, ch, row, 0)
        pltpu.sync_copy(val_v, out_hbm.at[wid])

    return sc_gather


def _tc_body(att_ref, tpos_ref, tneg_ref, w_ref, b_ref, h_ref, pos_ref, neg_ref):
    rs = att_ref.shape[2]
    tpos = tpos_ref[...]
    tv = lax.dot_general(
        tpos, w_ref[...], (((1,), (0,)), ((), ())),
        preferred_element_type=jnp.float32,
    )
    m = att_ref[0]
    b = b_ref[...]
    pos_cols = []
    neg_cols = []
    for g in range(rs):
        sl = slice(g * 128, (g + 1) * 128)
        col = m[:, g : g + 1]
        h_g = jnp.maximum(col * tv[sl] + b, 0.0)
        h_ref[sl, :] = h_g
        pos_cols.append(jnp.sum(h_g * tpos[sl], axis=1, keepdims=True))
        neg_cols.append(jnp.sum(h_g * tneg_ref[sl, :], axis=1, keepdims=True))
    pos_ref[0] = jnp.concatenate(pos_cols, axis=1)
    neg_ref[0] = jnp.concatenate(neg_cols, axis=1)


def kernel(h_e, t_pos_e, t_neg_e, h_indices, pos_t_indices, A_in, kernel, bias, update):
    del h_e, update
    B, D = t_pos_e.shape
    U = kernel.shape[1]
    n_cols = A_in.shape[1]

    R = 3200
    RS = R // 128
    NB = B // R
    flat = h_indices[0].astype(jnp.int32) * jnp.int32(n_cols) + pos_t_indices[0]
    flat = flat.reshape(NB, RS, 128).transpose(0, 2, 1).reshape(-1)
    ch = -(-B // (_NW * _CW))
    ch = -(-ch // 8) * 8
    total = _NW * ch * _CW
    pad = total - B
    if pad:
        flat = jnp.concatenate([flat, jnp.arange(pad, dtype=jnp.int32)])
    idx3 = flat.reshape(_NW, ch, _CW)
    att3 = _sc_gather_fn(ch)(A_in.reshape(-1), idx3)
    att_l = att3.reshape(-1)[:B].reshape(NB, 128, RS)

    h, pos, neg = pl.pallas_call(
        _tc_body,
        grid=(NB,),
        in_specs=[
            pl.BlockSpec((1, 128, RS), lambda i: (i, 0, 0)),
            pl.BlockSpec((R, D), lambda i: (i, 0)),
            pl.BlockSpec((R, D), lambda i: (i, 0)),
            pl.BlockSpec((D, U), lambda i: (0, 0)),
            pl.BlockSpec((1, U), lambda i: (0, 0)),
        ],
        out_specs=[
            pl.BlockSpec((R, U), lambda i: (i, 0)),
            pl.BlockSpec((1, 128, RS), lambda i: (i, 0, 0)),
            pl.BlockSpec((1, 128, RS), lambda i: (i, 0, 0)),
        ],
        out_shape=[
            jax.ShapeDtypeStruct((B, U), jnp.float32),
            jax.ShapeDtypeStruct((NB, 128, RS), jnp.float32),
            jax.ShapeDtypeStruct((NB, 128, RS), jnp.float32),
        ],
        compiler_params=pltpu.CompilerParams(
            dimension_semantics=("parallel",),
        ),
    )(att_l, t_pos_e, t_neg_e, kernel, bias.reshape(1, U))

    pos = pos.transpose(0, 2, 1).reshape(B)
    neg = neg.transpose(0, 2, 1).reshape(B)
    return h, pos, neg


# speedup vs baseline: 1.0400x; 1.0400x over previous
import functools

import jax
import jax.numpy as jnp
from jax import lax
from jax.experimental import pallas as pl
from jax.experimental.pallas import tpu as pltpu
from jax.experimental.pallas import tpu_sc as plsc

_NC = 2
_NS = 16
_NW = _NC * _NS

_CW = 128


def _sc_gather_fn(ch):
    mesh = plsc.VectorSubcoreMesh(core_axis_name="c", subcore_axis_name="s")

    @functools.partial(
        pl.kernel,
        mesh=mesh,
        out_type=jax.ShapeDtypeStruct((_NW, ch, _CW), jnp.float32),
        scratch_types=[
            pltpu.VMEM((ch, _CW), jnp.int32),
            pltpu.VMEM((ch, _CW), jnp.float32),
            pltpu.SemaphoreType.DMA,
        ],
    )
    def sc_gather(a_hbm, idx_hbm, out_hbm, idx_v, val_v, sem):
        wid = lax.axis_index("s") * _NC + lax.axis_index("c")
        pltpu.sync_copy(idx_hbm.at[wid], idx_v)

        copies = [
            pltpu.async_copy(
                a_hbm.at[idx_v[r, pl.ds(k * 16, 16)]],
                val_v.at[r, pl.ds(k * 16, 16)],
                sem,
            )
            for r in range(ch)
            for k in range(_CW // 16)
        ]
        for c in copies:
            c.wait()
        pltpu.sync_copy(val_v, out_hbm.at[wid])

    return sc_gather


def _tc_body(att_ref, tpos_ref, tneg_ref, w_ref, b_ref, h_ref, pos_ref, neg_ref):
    rs = att_ref.shape[2]
    tpos = tpos_ref[...]
    tv = lax.dot_general(
        tpos, w_ref[...], (((1,), (0,)), ((), ())),
        preferred_element_type=jnp.float32,
    )
    m = att_ref[0]
    b = b_ref[...]
    pos_cols = []
    neg_cols = []
    for g in range(rs):
        sl = slice(g * 128, (g + 1) * 128)
        col = m[:, g : g + 1]
        h_g = jnp.maximum(col * tv[sl] + b, 0.0)
        h_ref[sl, :] = h_g
        pos_cols.append(jnp.sum(h_g * tpos[sl], axis=1, keepdims=True))
        neg_cols.append(jnp.sum(h_g * tneg_ref[sl, :], axis=1, keepdims=True))
    pos_ref[0] = jnp.concatenate(pos_cols, axis=1)
    neg_ref[0] = jnp.concatenate(neg_cols, axis=1)


def kernel(h_e, t_pos_e, t_neg_e, h_indices, pos_t_indices, A_in, kernel, bias, update):
    del h_e, update
    B, D = t_pos_e.shape
    U = kernel.shape[1]
    n_cols = A_in.shape[1]

    R = 3200
    RS = R // 128
    NB = B // R
    flat = h_indices[0].astype(jnp.int32) * jnp.int32(n_cols) + pos_t_indices[0]
    flat = flat.reshape(NB, RS, 128).transpose(0, 2, 1).reshape(-1)
    ch = -(-B // (_NW * _CW))
    ch = -(-ch // 8) * 8
    total = _NW * ch * _CW
    pad = total - B
    if pad:
        flat = jnp.concatenate([flat, jnp.arange(pad, dtype=jnp.int32)])
    idx3 = flat.reshape(_NW, ch, _CW)
    att3 = _sc_gather_fn(ch)(A_in.reshape(-1), idx3)
    att_l = att3.reshape(-1)[:B].reshape(NB, 128, RS)

    h, pos, neg = pl.pallas_call(
        _tc_body,
        grid=(NB,),
        in_specs=[
            pl.BlockSpec((1, 128, RS), lambda i: (i, 0, 0)),
            pl.BlockSpec((R, D), lambda i: (i, 0)),
            pl.BlockSpec((R, D), lambda i: (i, 0)),
            pl.BlockSpec((D, U), lambda i: (0, 0)),
            pl.BlockSpec((1, U), lambda i: (0, 0)),
        ],
        out_specs=[
            pl.BlockSpec((R, U), lambda i: (i, 0)),
            pl.BlockSpec((1, 128, RS), lambda i: (i, 0, 0)),
            pl.BlockSpec((1, 128, RS), lambda i: (i, 0, 0)),
        ],
        out_shape=[
            jax.ShapeDtypeStruct((B, U), jnp.float32),
            jax.ShapeDtypeStruct((NB, 128, RS), jnp.float32),
            jax.ShapeDtypeStruct((NB, 128, RS), jnp.float32),
        ],
        compiler_params=pltpu.CompilerParams(
            dimension_semantics=("parallel",),
        ),
    )(att_l, t_pos_e, t_neg_e, kernel, bias.reshape(1, U))

    pos = pos.transpose(0, 2, 1).reshape(B)
    neg = neg.transpose(0, 2, 1).reshape(B)
    return h, pos, neg


# speedup vs baseline: 1.0551x; 1.0145x over previous
import functools

import jax
import jax.numpy as jnp
from jax import lax
from jax.experimental import pallas as pl
from jax.experimental.pallas import tpu as pltpu
from jax.experimental.pallas import tpu_sc as plsc

_NC = 2
_NS = 16
_NW = _NC * _NS

_CW = 128


def _sc_gather_fn(chw):
    mesh = plsc.VectorSubcoreMesh(core_axis_name="c", subcore_axis_name="s")

    @functools.partial(
        pl.kernel,
        mesh=mesh,
        out_type=jax.ShapeDtypeStruct((_NW * chw,), jnp.float32),
        scratch_types=[
            pltpu.VMEM((chw,), jnp.int32),
            pltpu.VMEM((chw,), jnp.float32),
            pltpu.SemaphoreType.DMA,
        ],
    )
    def sc_gather(a_hbm, idx_hbm, out_hbm, idx_v, val_v, sem):
        wid = lax.axis_index("s") * _NC + lax.axis_index("c")
        base = wid * chw
        pltpu.sync_copy(idx_hbm.at[pl.ds(base, chw)], idx_v)
        pltpu.async_copy(a_hbm.at[idx_v], val_v, sem).wait()
        pltpu.sync_copy(val_v, out_hbm.at[pl.ds(base, chw)])

    return sc_gather


def _tc_body(att_ref, tpos_ref, tneg_ref, w_ref, b_ref, h_ref, pos_ref, neg_ref):
    rs = att_ref.shape[2]
    tpos = tpos_ref[...]
    tv = lax.dot_general(
        tpos, w_ref[...], (((1,), (0,)), ((), ())),
        preferred_element_type=jnp.float32,
    )
    m = att_ref[0]
    b = b_ref[...]
    pos_cols = []
    neg_cols = []
    for g in range(rs):
        sl = slice(g * 128, (g + 1) * 128)
        col = m[:, g : g + 1]
        h_g = jnp.maximum(col * tv[sl] + b, 0.0)
        h_ref[sl, :] = h_g
        pos_cols.append(jnp.sum(h_g * tpos[sl], axis=1, keepdims=True))
        neg_cols.append(jnp.sum(h_g * tneg_ref[sl, :], axis=1, keepdims=True))
    pos_ref[0] = jnp.concatenate(pos_cols, axis=1)
    neg_ref[0] = jnp.concatenate(neg_cols, axis=1)


def kernel(h_e, t_pos_e, t_neg_e, h_indices, pos_t_indices, A_in, kernel, bias, update):
    del h_e, update
    B, D = t_pos_e.shape
    U = kernel.shape[1]
    n_cols = A_in.shape[1]

    R = 3200
    RS = R // 128
    NB = B // R
    flat = h_indices[0].astype(jnp.int32) * jnp.int32(n_cols) + pos_t_indices[0]
    flat = flat.reshape(NB, RS, 128).transpose(0, 2, 1).reshape(-1)
    chw = -(-B // (_NW * 8)) * 8
    total = _NW * chw
    pad = total - B
    if pad:
        flat = jnp.concatenate([flat, jnp.arange(pad, dtype=jnp.int32)])
    att3 = _sc_gather_fn(chw)(A_in.reshape(-1), flat)
    att_l = att3[:B].reshape(NB, 128, RS)

    h, pos, neg = pl.pallas_call(
        _tc_body,
        grid=(NB,),
        in_specs=[
            pl.BlockSpec((1, 128, RS), lambda i: (i, 0, 0)),
            pl.BlockSpec((R, D), lambda i: (i, 0)),
            pl.BlockSpec((R, D), lambda i: (i, 0)),
            pl.BlockSpec((D, U), lambda i: (0, 0)),
            pl.BlockSpec((1, U), lambda i: (0, 0)),
        ],
        out_specs=[
            pl.BlockSpec((R, U), lambda i: (i, 0)),
            pl.BlockSpec((1, 128, RS), lambda i: (i, 0, 0)),
            pl.BlockSpec((1, 128, RS), lambda i: (i, 0, 0)),
        ],
        out_shape=[
            jax.ShapeDtypeStruct((B, U), jnp.float32),
            jax.ShapeDtypeStruct((NB, 128, RS), jnp.float32),
            jax.ShapeDtypeStruct((NB, 128, RS), jnp.float32),
        ],
        compiler_params=pltpu.CompilerParams(
            dimension_semantics=("parallel",),
        ),
    )(att_l, t_pos_e, t_neg_e, kernel, bias.reshape(1, U))

    pos = pos.transpose(0, 2, 1).reshape(B)
    neg = neg.transpose(0, 2, 1).reshape(B)
    return h, pos, neg
